# baseline (speedup 1.0000x reference)
import functools

import jax
import jax.numpy as jnp
from jax.experimental import pallas as pl
from jax.experimental.pallas import tpu as pltpu

_K = 1024
_D = 32
_TILE = 512
_SUB = 128


def _vq_kernel(x_ref, a_ref, b_ref, iota_ref, w_ref, out_ref, loss_ref,
               counts_ref, sq_ref):
    i = pl.program_id(0)
    nsteps = pl.num_programs(0)

    @pl.when(i == 0)
    def _init():
        counts_ref[...] = jnp.zeros_like(counts_ref)
        sq_ref[...] = jnp.zeros_like(sq_ref)

    w = w_ref[...]
    iota = iota_ref[...]
    b = b_ref[...]

    nsub = _TILE // _SUB

    def _mm(k):
        x_s = x_ref[pl.ds(k * _SUB, _SUB), :]
        return jax.lax.dot_general(x_s * -2.0, w,
                                   dimension_numbers=(((1,), (1,)), ((), ())))

    csums = []
    sqs = []
    c2_next = _mm(0)
    for k in range(nsub):
        c2 = c2_next
        if k + 1 < nsub:
            c2_next = _mm(k + 1)
        x_s = x_ref[pl.ds(k * _SUB, _SUB), :]
        a_s = a_ref[pl.ds(k * _SUB, _SUB), :]
        d = (a_s + b) + c2
        m = jnp.min(d, axis=1, keepdims=True)
        sel = jnp.where(d == m, iota, float(_K))
        amin = jnp.min(sel, axis=1, keepdims=True)
        onehot = (iota == amin).astype(jnp.float32)
        q = jax.lax.dot_general(onehot, w,
                                dimension_numbers=(((1,), (0,)), ((), ())))
        diff = q - x_s
        out_ref[pl.ds(k * _SUB, _SUB), :] = x_s + diff
        ones_row = jnp.ones((1, _SUB), jnp.float32)
        csums.append(jax.lax.dot_general(
            ones_row, onehot, dimension_numbers=(((1,), (0,)), ((), ()))))
        sqs.append(jnp.sum(diff * diff, axis=0, keepdims=True))

    counts_ref[...] = counts_ref[...] + sum(csums)
    sq_ref[...] = sq_ref[...] + sum(sqs)

    @pl.when(i == nsteps - 1)
    def _finalize():
        n_total = nsteps * _TILE
        p = counts_ref[...] * (1.0 / n_total)
        mse = jnp.sum(sq_ref[...]) / (n_total * _D)
        loss = mse + 0.25 * mse
        entropy = -jnp.sum(p * jnp.log(p + 1e-10))
        div = jnp.sum((p - 1.0 / _K) ** 2)
        kl = jnp.sum(p * jnp.log(p * float(_K) + 1e-10))
        loss_ref[0, 0] = ((loss - entropy) + div) + kl


@functools.partial(jax.jit)
def kernel(x, W):
    flat_x = x.reshape(-1, _D)
    n = flat_x.shape[0]
    a = jnp.sum(flat_x ** 2, axis=1, keepdims=True)
    b = jnp.sum(W ** 2, axis=1)[None, :]
    iota = jnp.arange(_K, dtype=jnp.float32)[None, :]
    out_q, out_loss = pl.pallas_call(
        _vq_kernel,
        grid=(n // _TILE,),
        in_specs=[
            pl.BlockSpec((_TILE, _D), lambda i: (i, 0)),
            pl.BlockSpec((_TILE, 1), lambda i: (i, 0)),
            pl.BlockSpec((1, _K), lambda i: (0, 0)),
            pl.BlockSpec((1, _K), lambda i: (0, 0)),
            pl.BlockSpec((_K, _D), lambda i: (0, 0)),
        ],
        out_specs=[
            pl.BlockSpec((_TILE, _D), lambda i: (i, 0)),
            pl.BlockSpec(memory_space=pltpu.SMEM),
        ],
        out_shape=[
            jax.ShapeDtypeStruct((n, _D), jnp.float32),
            jax.ShapeDtypeStruct((1, 1), jnp.float32),
        ],
        scratch_shapes=[
            pltpu.VMEM((1, _K), jnp.float32),
            pltpu.VMEM((1, _D), jnp.float32),
        ],
    )(flat_x, a, b, iota, W)
    return out_q.reshape(x.shape), out_loss.reshape(())


# speedup vs baseline: 1.0888x; 1.0888x over previous
import functools

import jax
import jax.numpy as jnp
from jax.experimental import pallas as pl
from jax.experimental.pallas import tpu as pltpu

_K = 1024
_D = 32
_TILE = 1024
_SUB = 128


def _vq_kernel(x_ref, a_ref, b_ref, iota_ref, w_ref, out_ref, loss_ref,
               counts_ref, sq_ref):
    i = pl.program_id(0)
    nsteps = pl.num_programs(0)

    @pl.when(i == 0)
    def _init():
        counts_ref[...] = jnp.zeros_like(counts_ref)
        sq_ref[...] = jnp.zeros_like(sq_ref)

    w = w_ref[...]
    iota = iota_ref[...]
    b = b_ref[...]

    nsub = _TILE // _SUB

    def _mm(k):
        x_s = x_ref[pl.ds(k * _SUB, _SUB), :]
        return jax.lax.dot_general(x_s * -2.0, w,
                                   dimension_numbers=(((1,), (1,)), ((), ())))

    csums = []
    sqs = []
    c2_next = _mm(0)
    for k in range(nsub):
        c2 = c2_next
        if k + 1 < nsub:
            c2_next = _mm(k + 1)
        x_s = x_ref[pl.ds(k * _SUB, _SUB), :]
        a_s = a_ref[pl.ds(k * _SUB, _SUB), :]
        d = (a_s + b) + c2
        m = jnp.min(d, axis=1, keepdims=True)
        sel = jnp.where(d == m, iota, float(_K))
        amin = jnp.min(sel, axis=1, keepdims=True)
        onehot = (iota == amin).astype(jnp.float32)
        q = jax.lax.dot_general(onehot, w,
                                dimension_numbers=(((1,), (0,)), ((), ())))
        diff = q - x_s
        out_ref[pl.ds(k * _SUB, _SUB), :] = x_s + diff
        ones_row = jnp.ones((1, _SUB), jnp.float32)
        csums.append(jax.lax.dot_general(
            ones_row, onehot, dimension_numbers=(((1,), (0,)), ((), ()))))
        sqs.append(jnp.sum(diff * diff, axis=0, keepdims=True))

    counts_ref[...] = counts_ref[...] + sum(csums)
    sq_ref[...] = sq_ref[...] + sum(sqs)

    @pl.when(i == nsteps - 1)
    def _finalize():
        n_total = nsteps * _TILE
        p = counts_ref[...] * (1.0 / n_total)
        mse = jnp.sum(sq_ref[...]) / (n_total * _D)
        loss = mse + 0.25 * mse
        entropy = -jnp.sum(p * jnp.log(p + 1e-10))
        div = jnp.sum((p - 1.0 / _K) ** 2)
        kl = jnp.sum(p * jnp.log(p * float(_K) + 1e-10))
        loss_ref[0, 0] = ((loss - entropy) + div) + kl


@functools.partial(jax.jit)
def kernel(x, W):
    flat_x = x.reshape(-1, _D)
    n = flat_x.shape[0]
    a = jnp.sum(flat_x ** 2, axis=1, keepdims=True)
    b = jnp.sum(W ** 2, axis=1)[None, :]
    iota = jnp.arange(_K, dtype=jnp.float32)[None, :]
    out_q, out_loss = pl.pallas_call(
        _vq_kernel,
        grid=(n // _TILE,),
        in_specs=[
            pl.BlockSpec((_TILE, _D), lambda i: (i, 0)),
            pl.BlockSpec((_TILE, 1), lambda i: (i, 0)),
            pl.BlockSpec((1, _K), lambda i: (0, 0)),
            pl.BlockSpec((1, _K), lambda i: (0, 0)),
            pl.BlockSpec((_K, _D), lambda i: (0, 0)),
        ],
        out_specs=[
            pl.BlockSpec((_TILE, _D), lambda i: (i, 0)),
            pl.BlockSpec(memory_space=pltpu.SMEM),
        ],
        out_shape=[
            jax.ShapeDtypeStruct((n, _D), jnp.float32),
            jax.ShapeDtypeStruct((1, 1), jnp.float32),
        ],
        scratch_shapes=[
            pltpu.VMEM((1, _K), jnp.float32),
            pltpu.VMEM((1, _D), jnp.float32),
        ],
    )(flat_x, a, b, iota, W)
    return out_q.reshape(x.shape), out_loss.reshape(())


# speedup vs baseline: 1.1443x; 1.0510x over previous
import functools

import jax
import jax.numpy as jnp
from jax.experimental import pallas as pl
from jax.experimental.pallas import tpu as pltpu

_K = 1024
_D = 32
_TILE = 2048
_SUB = 128


def _vq_kernel(x_ref, a_ref, b_ref, iota_ref, w_ref, out_ref, loss_ref,
               counts_ref, sq_ref):
    i = pl.program_id(0)
    nsteps = pl.num_programs(0)

    @pl.when(i == 0)
    def _init():
        counts_ref[...] = jnp.zeros_like(counts_ref)
        sq_ref[...] = jnp.zeros_like(sq_ref)

    w = w_ref[...]
    iota = iota_ref[...]
    b = b_ref[...]

    nsub = _TILE // _SUB

    def _mm(k):
        x_s = x_ref[pl.ds(k * _SUB, _SUB), :]
        return jax.lax.dot_general(x_s * -2.0, w,
                                   dimension_numbers=(((1,), (1,)), ((), ())))

    csums = []
    sqs = []
    c2_next = _mm(0)
    for k in range(nsub):
        c2 = c2_next
        if k + 1 < nsub:
            c2_next = _mm(k + 1)
        x_s = x_ref[pl.ds(k * _SUB, _SUB), :]
        a_s = a_ref[pl.ds(k * _SUB, _SUB), :]
        d = (a_s + b) + c2
        m = jnp.min(d, axis=1, keepdims=True)
        sel = jnp.where(d == m, iota, float(_K))
        amin = jnp.min(sel, axis=1, keepdims=True)
        onehot = (iota == amin).astype(jnp.float32)
        q = jax.lax.dot_general(onehot, w,
                                dimension_numbers=(((1,), (0,)), ((), ())))
        diff = q - x_s
        out_ref[pl.ds(k * _SUB, _SUB), :] = x_s + diff
        ones_row = jnp.ones((1, _SUB), jnp.float32)
        csums.append(jax.lax.dot_general(
            ones_row, onehot, dimension_numbers=(((1,), (0,)), ((), ()))))
        sqs.append(jnp.sum(diff * diff, axis=0, keepdims=True))

    counts_ref[...] = counts_ref[...] + sum(csums)
    sq_ref[...] = sq_ref[...] + sum(sqs)

    @pl.when(i == nsteps - 1)
    def _finalize():
        n_total = nsteps * _TILE
        p = counts_ref[...] * (1.0 / n_total)
        mse = jnp.sum(sq_ref[...]) / (n_total * _D)
        loss = mse + 0.25 * mse
        entropy = -jnp.sum(p * jnp.log(p + 1e-10))
        div = jnp.sum((p - 1.0 / _K) ** 2)
        kl = jnp.sum(p * jnp.log(p * float(_K) + 1e-10))
        loss_ref[0, 0] = ((loss - entropy) + div) + kl


@functools.partial(jax.jit)
def kernel(x, W):
    flat_x = x.reshape(-1, _D)
    n = flat_x.shape[0]
    a = jnp.sum(flat_x ** 2, axis=1, keepdims=True)
    b = jnp.sum(W ** 2, axis=1)[None, :]
    iota = jnp.arange(_K, dtype=jnp.float32)[None, :]
    out_q, out_loss = pl.pallas_call(
        _vq_kernel,
        grid=(n // _TILE,),
        in_specs=[
            pl.BlockSpec((_TILE, _D), lambda i: (i, 0)),
            pl.BlockSpec((_TILE, 1), lambda i: (i, 0)),
            pl.BlockSpec((1, _K), lambda i: (0, 0)),
            pl.BlockSpec((1, _K), lambda i: (0, 0)),
            pl.BlockSpec((_K, _D), lambda i: (0, 0)),
        ],
        out_specs=[
            pl.BlockSpec((_TILE, _D), lambda i: (i, 0)),
            pl.BlockSpec(memory_space=pltpu.SMEM),
        ],
        out_shape=[
            jax.ShapeDtypeStruct((n, _D), jnp.float32),
            jax.ShapeDtypeStruct((1, 1), jnp.float32),
        ],
        scratch_shapes=[
            pltpu.VMEM((1, _K), jnp.float32),
            pltpu.VMEM((1, _D), jnp.float32),
        ],
    )(flat_x, a, b, iota, W)
    return out_q.reshape(x.shape), out_loss.reshape(())


# speedup vs baseline: 1.1693x; 1.0219x over previous
import functools

import jax
import jax.numpy as jnp
from jax.experimental import pallas as pl
from jax.experimental.pallas import tpu as pltpu

_K = 1024
_D = 32
_TILE = 4096
_SUB = 128


def _vq_kernel(x_ref, a_ref, b_ref, iota_ref, w_ref, out_ref, loss_ref,
               counts_ref, sq_ref):
    i = pl.program_id(0)
    nsteps = pl.num_programs(0)

    @pl.when(i == 0)
    def _init():
        counts_ref[...] = jnp.zeros_like(counts_ref)
        sq_ref[...] = jnp.zeros_like(sq_ref)

    w = w_ref[...]
    iota = iota_ref[...]
    b = b_ref[...]

    nsub = _TILE // _SUB

    def _mm(k):
        x_s = x_ref[pl.ds(k * _SUB, _SUB), :]
        return jax.lax.dot_general(x_s * -2.0, w,
                                   dimension_numbers=(((1,), (1,)), ((), ())))

    csums = []
    sqs = []
    c2_next = _mm(0)
    for k in range(nsub):
        c2 = c2_next
        if k + 1 < nsub:
            c2_next = _mm(k + 1)
        x_s = x_ref[pl.ds(k * _SUB, _SUB), :]
        a_s = a_ref[pl.ds(k * _SUB, _SUB), :]
        d = (a_s + b) + c2
        m = jnp.min(d, axis=1, keepdims=True)
        sel = jnp.where(d == m, iota, float(_K))
        amin = jnp.min(sel, axis=1, keepdims=True)
        onehot = (iota == amin).astype(jnp.float32)
        q = jax.lax.dot_general(onehot, w,
                                dimension_numbers=(((1,), (0,)), ((), ())))
        diff = q - x_s
        out_ref[pl.ds(k * _SUB, _SUB), :] = x_s + diff
        ones_row = jnp.ones((1, _SUB), jnp.float32)
        csums.append(jax.lax.dot_general(
            ones_row, onehot, dimension_numbers=(((1,), (0,)), ((), ()))))
        sqs.append(jnp.sum(diff * diff, axis=0, keepdims=True))

    counts_ref[...] = counts_ref[...] + sum(csums)
    sq_ref[...] = sq_ref[...] + sum(sqs)

    @pl.when(i == nsteps - 1)
    def _finalize():
        n_total = nsteps * _TILE
        p = counts_ref[...] * (1.0 / n_total)
        mse = jnp.sum(sq_ref[...]) / (n_total * _D)
        loss = mse + 0.25 * mse
        entropy = -jnp.sum(p * jnp.log(p + 1e-10))
        div = jnp.sum((p - 1.0 / _K) ** 2)
        kl = jnp.sum(p * jnp.log(p * float(_K) + 1e-10))
        loss_ref[0, 0] = ((loss - entropy) + div) + kl


@functools.partial(jax.jit)
def kernel(x, W):
    flat_x = x.reshape(-1, _D)
    n = flat_x.shape[0]
    a = jnp.sum(flat_x ** 2, axis=1, keepdims=True)
    b = jnp.sum(W ** 2, axis=1)[None, :]
    iota = jnp.arange(_K, dtype=jnp.float32)[None, :]
    out_q, out_loss = pl.pallas_call(
        _vq_kernel,
        grid=(n // _TILE,),
        in_specs=[
            pl.BlockSpec((_TILE, _D), lambda i: (i, 0)),
            pl.BlockSpec((_TILE, 1), lambda i: (i, 0)),
            pl.BlockSpec((1, _K), lambda i: (0, 0)),
            pl.BlockSpec((1, _K), lambda i: (0, 0)),
            pl.BlockSpec((_K, _D), lambda i: (0, 0)),
        ],
        out_specs=[
            pl.BlockSpec((_TILE, _D), lambda i: (i, 0)),
            pl.BlockSpec(memory_space=pltpu.SMEM),
        ],
        out_shape=[
            jax.ShapeDtypeStruct((n, _D), jnp.float32),
            jax.ShapeDtypeStruct((1, 1), jnp.float32),
        ],
        scratch_shapes=[
            pltpu.VMEM((1, _K), jnp.float32),
            pltpu.VMEM((1, _D), jnp.float32),
        ],
    )(flat_x, a, b, iota, W)
    return out_q.reshape(x.shape), out_loss.reshape(())


# speedup vs baseline: 1.2673x; 1.0838x over previous
import functools

import jax
import jax.numpy as jnp
from jax.experimental import pallas as pl
from jax.experimental.pallas import tpu as pltpu

_K = 1024
_D = 32
_TILE = 4096
_SUB = 128


def _vq_kernel(x_ref, b_ref, iota_ref, w_ref, out_ref, loss_ref,
               counts_ref, sq_ref):
    i = pl.program_id(0)
    nsteps = pl.num_programs(0)

    @pl.when(i == 0)
    def _init():
        counts_ref[...] = jnp.zeros_like(counts_ref)
        sq_ref[...] = jnp.zeros_like(sq_ref)

    w = w_ref[...]
    iota = iota_ref[...]
    b = b_ref[...]

    nsub = _TILE // _SUB

    def _mm(k):
        x_s = x_ref[pl.ds(k * _SUB, _SUB), :]
        return jax.lax.dot_general(x_s * -2.0, w,
                                   dimension_numbers=(((1,), (1,)), ((), ())))

    csums = []
    sqs = []
    c2_next = _mm(0)
    for k in range(nsub):
        c2 = c2_next
        if k + 1 < nsub:
            c2_next = _mm(k + 1)
        x_s = x_ref[pl.ds(k * _SUB, _SUB), :]
        a_s = jnp.sum(x_s * x_s, axis=1, keepdims=True)
        d = (a_s + b) + c2
        m = jnp.min(d, axis=1, keepdims=True)
        sel = jnp.where(d == m, iota, float(_K))
        amin = jnp.min(sel, axis=1, keepdims=True)
        onehot = (iota == amin).astype(jnp.float32)
        q = jax.lax.dot_general(onehot, w,
                                dimension_numbers=(((1,), (0,)), ((), ())))
        diff = q - x_s
        out_ref[pl.ds(k * _SUB, _SUB), :] = x_s + diff
        ones_row = jnp.ones((1, _SUB), jnp.float32)
        csums.append(jax.lax.dot_general(
            ones_row, onehot, dimension_numbers=(((1,), (0,)), ((), ()))))
        sqs.append(jnp.sum(diff * diff, axis=0, keepdims=True))

    counts_ref[...] = counts_ref[...] + sum(csums)
    sq_ref[...] = sq_ref[...] + sum(sqs)

    @pl.when(i == nsteps - 1)
    def _finalize():
        n_total = nsteps * _TILE
        p = counts_ref[...] * (1.0 / n_total)
        mse = jnp.sum(sq_ref[...]) / (n_total * _D)
        loss = mse + 0.25 * mse
        entropy = -jnp.sum(p * jnp.log(p + 1e-10))
        div = jnp.sum((p - 1.0 / _K) ** 2)
        kl = jnp.sum(p * jnp.log(p * float(_K) + 1e-10))
        loss_ref[0, 0] = ((loss - entropy) + div) + kl


@functools.partial(jax.jit)
def kernel(x, W):
    flat_x = x.reshape(-1, _D)
    n = flat_x.shape[0]
    b = jnp.sum(W ** 2, axis=1)[None, :]
    iota = jnp.arange(_K, dtype=jnp.float32)[None, :]
    out_q, out_loss = pl.pallas_call(
        _vq_kernel,
        grid=(n // _TILE,),
        in_specs=[
            pl.BlockSpec((_TILE, _D), lambda i: (i, 0)),
            pl.BlockSpec((1, _K), lambda i: (0, 0)),
            pl.BlockSpec((1, _K), lambda i: (0, 0)),
            pl.BlockSpec((_K, _D), lambda i: (0, 0)),
        ],
        out_specs=[
            pl.BlockSpec((_TILE, _D), lambda i: (i, 0)),
            pl.BlockSpec(memory_space=pltpu.SMEM),
        ],
        out_shape=[
            jax.ShapeDtypeStruct((n, _D), jnp.float32),
            jax.ShapeDtypeStruct((1, 1), jnp.float32),
        ],
        scratch_shapes=[
            pltpu.VMEM((1, _K), jnp.float32),
            pltpu.VMEM((1, _D), jnp.float32),
        ],
    )(flat_x, b, iota, W)
    return out_q.reshape(x.shape), out_loss.reshape(())


# speedup vs baseline: 1.2709x; 1.0029x over previous
import functools

import jax
import jax.numpy as jnp
from jax.experimental import pallas as pl
from jax.experimental.pallas import tpu as pltpu

_K = 1024
_D = 32
_TILE = 4096
_SUB = 128


def _vq_kernel(x_ref, w_ref, out_ref, loss_ref,
               counts_ref, sq_ref):
    i = pl.program_id(0)
    nsteps = pl.num_programs(0)

    @pl.when(i == 0)
    def _init():
        counts_ref[...] = jnp.zeros_like(counts_ref)
        sq_ref[...] = jnp.zeros_like(sq_ref)

    w = w_ref[...]
    iota = jax.lax.broadcasted_iota(jnp.int32, (1, _K), 1).astype(jnp.float32)
    b_col = jnp.sum(w * w, axis=1, keepdims=True)
    b = jnp.swapaxes(b_col, 0, 1)

    nsub = _TILE // _SUB

    def _mm(k):
        x_s = x_ref[pl.ds(k * _SUB, _SUB), :]
        return jax.lax.dot_general(x_s * -2.0, w,
                                   dimension_numbers=(((1,), (1,)), ((), ())))

    csums = []
    sqs = []
    c2_next = _mm(0)
    for k in range(nsub):
        c2 = c2_next
        if k + 1 < nsub:
            c2_next = _mm(k + 1)
        x_s = x_ref[pl.ds(k * _SUB, _SUB), :]
        a_s = jnp.sum(x_s * x_s, axis=1, keepdims=True)
        d = (a_s + b) + c2
        m = jnp.min(d, axis=1, keepdims=True)
        sel = jnp.where(d == m, iota, float(_K))
        amin = jnp.min(sel, axis=1, keepdims=True)
        onehot = (iota == amin).astype(jnp.float32)
        q = jax.lax.dot_general(onehot, w,
                                dimension_numbers=(((1,), (0,)), ((), ())))
        diff = q - x_s
        out_ref[pl.ds(k * _SUB, _SUB), :] = x_s + diff
        ones_row = jnp.ones((1, _SUB), jnp.float32)
        csums.append(jax.lax.dot_general(
            ones_row, onehot, dimension_numbers=(((1,), (0,)), ((), ()))))
        sqs.append(jnp.sum(diff * diff, axis=0, keepdims=True))

    counts_ref[...] = counts_ref[...] + sum(csums)
    sq_ref[...] = sq_ref[...] + sum(sqs)

    @pl.when(i == nsteps - 1)
    def _finalize():
        n_total = nsteps * _TILE
        p = counts_ref[...] * (1.0 / n_total)
        mse = jnp.sum(sq_ref[...]) / (n_total * _D)
        loss = mse + 0.25 * mse
        entropy = -jnp.sum(p * jnp.log(p + 1e-10))
        div = jnp.sum((p - 1.0 / _K) ** 2)
        kl = jnp.sum(p * jnp.log(p * float(_K) + 1e-10))
        loss_ref[0, 0] = ((loss - entropy) + div) + kl


@functools.partial(jax.jit)
def kernel(x, W):
    flat_x = x.reshape(-1, _D)
    n = flat_x.shape[0]
    out_q, out_loss = pl.pallas_call(
        _vq_kernel,
        grid=(n // _TILE,),
        in_specs=[
            pl.BlockSpec((_TILE, _D), lambda i: (i, 0)),
            pl.BlockSpec((_K, _D), lambda i: (0, 0)),
        ],
        out_specs=[
            pl.BlockSpec((_TILE, _D), lambda i: (i, 0)),
            pl.BlockSpec(memory_space=pltpu.SMEM),
        ],
        out_shape=[
            jax.ShapeDtypeStruct((n, _D), jnp.float32),
            jax.ShapeDtypeStruct((1, 1), jnp.float32),
        ],
        scratch_shapes=[
            pltpu.VMEM((1, _K), jnp.float32),
            pltpu.VMEM((1, _D), jnp.float32),
        ],
    )(flat_x, W)
    return out_q.reshape(x.shape), out_loss.reshape(())
